# baseline (speedup 1.0000x reference)
import functools

import jax
import jax.numpy as jnp
import numpy as np
from jax import lax
from jax.experimental import pallas as pl
from jax.experimental.pallas import tpu as pltpu
from jax.experimental.pallas import tpu_sc as plsc

H = 8
U = 64
HU = H * U
CW = HU + 16
L = 16
NC = 2
NS = 16
RN = 1280
NRANGE = 8
NSLOT = 4


def _vgather(v, idx):
    dnums = lax.GatherDimensionNumbers(
        offset_dims=(), collapsed_slice_dims=(0,), start_index_map=(0,))
    return lax.gather(v, idx[:, None], dnums, (1,),
                      mode=lax.GatherScatterMode.PROMISE_IN_BOUNDS)


def _proj_kernel(x_ref, w_ref, a1_ref, a2_ref, xp_ref, ft_ref, fs_ref):
    xp = jnp.dot(x_ref[...], w_ref[...], preferred_element_type=jnp.float32)
    xp_ref[...] = xp
    ft_ref[...] = jnp.dot(xp, a1_ref[...], preferred_element_type=jnp.float32)
    fs_ref[...] = jnp.dot(xp, a2_ref[...], preferred_element_type=jnp.float32)


def _sc_body(N, E, tgt_hbm, src_hbm, ft_hbm, fs_hbm, cs_hbm, bias_hbm,
             out_hbm, acc_sp, tgt_v, src_v, lsrc, ltgtl,
             tix, ftr, fsr, xbf, prod, zrow, bias_v,
             gsem_f, gsem_s, gsem_c, ssem):
    ept = E // NS
    chmax = ept // L
    c = lax.axis_index("c")
    t = lax.axis_index("s")
    lanes = lax.iota(jnp.int32, L)
    zi = jnp.zeros((L,), jnp.int32)
    zf = jnp.zeros((L,), jnp.float32)
    hidx = [jnp.full((L,), h, jnp.int32) for h in range(H)]

    ebase = t * ept
    pltpu.sync_copy(tgt_hbm.at[pl.ds(ebase, ept)], tgt_v)
    pltpu.sync_copy(src_hbm.at[pl.ds(ebase, ept)], src_v)
    pltpu.sync_copy(bias_hbm, bias_v)

    def z_body(e, carry):
        for j in range(CW // L):
            zrow[e, pl.ds(j * L, L)] = zf
        return carry
    lax.fori_loop(0, L, z_body, jnp.int32(0))

    sbase = t * (RN // NS)

    def range_body(r, rcarry):
        lo = (c * (NRANGE // NC) + r) * RN

        def issue_gathers(k, s):
            tix[s, :] = ltgtl[k, :] + lo
            pltpu.async_copy(ft_hbm.at[tix.at[s]], ftr.at[s], gsem_f[s])
            pltpu.async_copy(fs_hbm.at[lsrc.at[k]], fsr.at[s], gsem_s[s])
            pltpu.async_copy(cs_hbm.at[lsrc.at[k]], xbf.at[s], gsem_c[s])

        def zero_body(b, carry):
            pltpu.sync_copy(zrow, acc_sp.at[pl.ds(sbase + b * L, L)])
            return carry
        lax.fori_loop(0, RN // NS // L, zero_body, jnp.int32(0))
        plsc.subcore_barrier()

        @plsc.parallel_loop(0, chmax, 1, unroll=2, carry=jnp.int32(0))
        def scan_body(ci, cnt):
            off = pl.multiple_of(ci * L, L)
            tg = tgt_v[pl.ds(off, L)]
            sr = src_v[pl.ds(off, L)]
            tl = tg - lo
            m = (tl >= 0) & (tl < RN)
            pos = cnt + plsc.cumsum(m.astype(jnp.int32)) - 1
            row = lax.shift_right_logical(pos, 4)
            col = lax.bitwise_and(pos, 15)
            plsc.store_scatter(lsrc, [row, col], sr, mask=m)
            plsc.store_scatter(ltgtl, [row, col], tl, mask=m)
            return cnt + jnp.max(plsc.all_reduce_population_count(m))

        cnt = scan_body

        nch = lax.shift_right_logical(cnt + 15, 4)
        nchp = ((nch + NSLOT - 1) // NSLOT) * NSLOT

        def tz_body(b, carry):
            pos2 = cnt + b * L + lanes
            mtz = pos2 < (nchp + 2) * L
            plsc.store_scatter(
                lsrc, [lax.shift_right_logical(pos2, 4),
                       lax.bitwise_and(pos2, 15)], zi, mask=mtz)
            plsc.store_scatter(
                ltgtl, [lax.shift_right_logical(pos2, 4),
                        lax.bitwise_and(pos2, 15)], zi, mask=mtz)
            return carry
        lax.fori_loop(0, NSLOT + 3, tz_body, jnp.int32(0))

        issue_gathers(jnp.int32(0), 0)
        issue_gathers(jnp.int32(1), 1)

        def trip_body(m, carry):
            for b in range(NSLOT):
                k = m * NSLOT + b
                s = b
                sn = (b + 2) % NSLOT

                ps = b % 2

                @pl.when(k >= 2)
                def _():
                    pltpu.make_async_copy(prod.at[ps], acc_sp.at[ltgtl.at[k]],
                                          ssem[ps]).wait()
                issue_gathers(k + 2, sn)
                pltpu.make_async_copy(ft_hbm.at[tix.at[s]], ftr.at[s],
                                      gsem_f[s]).wait()
                pltpu.make_async_copy(fs_hbm.at[lsrc.at[k]], fsr.at[s],
                                      gsem_s[s]).wait()
                pltpu.make_async_copy(cs_hbm.at[lsrc.at[k]], xbf.at[s],
                                      gsem_c[s]).wait()

                base = k * L

                @plsc.parallel_loop(0, L, 1, unroll=4)
                def p_body(e):
                    rowv = ftr[s, e, :] + fsr[s, e, :]
                    sv = jnp.where(rowv > 0, rowv, 0.2 * rowv)
                    p = jnp.exp(sv)
                    p = jnp.where(base + e < cnt, p, 0.0)
                    prod[ps, e, pl.ds(HU, L)] = p
                    for j in range(HU // (2 * L)):
                        xv = xbf[s, e, pl.ds(j * 2 * L, 2 * L)]
                        va, vb = plsc.unpack(
                            xv, format=plsc.PackFormat.INTERLEAVED,
                            preferred_element_type=jnp.float32)
                        w = _vgather(p, hidx[j // 2])
                        prod[ps, e, pl.ds(j * 2 * L, L)] = va * w
                        prod[ps, e, pl.ds(j * 2 * L + L, L)] = vb * w
                pltpu.async_copy(prod.at[ps], acc_sp.at[ltgtl.at[k]],
                                 ssem[ps], add=True)
            return carry

        nts = nchp // NSLOT
        lax.fori_loop(0, nts, trip_body, jnp.int32(0))

        for ds_ in range(2):
            pltpu.make_async_copy(ft_hbm.at[tix.at[ds_]], ftr.at[ds_],
                                  gsem_f[ds_]).wait()
            pltpu.make_async_copy(fs_hbm.at[lsrc.at[0]], fsr.at[ds_],
                                  gsem_s[ds_]).wait()
            pltpu.make_async_copy(cs_hbm.at[lsrc.at[0]], xbf.at[ds_],
                                  gsem_c[ds_]).wait()

        @pl.when(nchp > 0)
        def _():
            pltpu.make_async_copy(prod.at[0], acc_sp.at[ltgtl.at[0]],
                                  ssem[0]).wait()
            pltpu.make_async_copy(prod.at[1], acc_sp.at[ltgtl.at[0]],
                                  ssem[1]).wait()
        plsc.subcore_barrier()

        def ep_body(b, carry):
            bstart = sbase + b * L
            gs = lo + bstart

            @pl.when(gs < N)
            def _():
                pltpu.sync_copy(acc_sp.at[pl.ds(bstart, L)], prod.at[0])

                @plsc.parallel_loop(0, L, 1, unroll=4)
                def row_body(e):
                    srow = prod[0, e, pl.ds(HU, L)]
                    for h in range(H):
                        sv = _vgather(srow, hidx[h]) + 1e-7
                        inv = 1.0 / sv
                        for q in range(HU // L // H):
                            j = h * (HU // L // H) + q
                            z = (prod[0, e, pl.ds(j * L, L)] * inv
                                 + bias_v[pl.ds(j * L, L)])
                            prod[0, e, pl.ds(j * L, L)] = jnp.where(
                                z > 0, z, jnp.exp(z) - 1.0)
                pltpu.sync_copy(prod.at[0], out_hbm.at[pl.ds(gs, L)])
            return carry
        lax.fori_loop(0, RN // NS // L, ep_body, jnp.int32(0))
        plsc.subcore_barrier()
        return rcarry
    lax.fori_loop(0, NRANGE // NC, range_body, jnp.int32(0))


def kernel(x, edges, kernel, kernel_attention1, kernel_attention2, bias,
           training):
    N, D = x.shape
    E = edges.shape[0]

    eye = jnp.eye(H, dtype=x.dtype)
    a1 = (kernel_attention1[0][:, :, None] * eye[:, None, :]).reshape(HU, H)
    a2 = (kernel_attention2[0][:, :, None] * eye[:, None, :]).reshape(HU, H)
    pad = jnp.zeros((HU, L - H), jnp.float32)
    a1p = jnp.concatenate([a1, pad], axis=1)
    a2p = jnp.concatenate([a2, pad], axis=1)

    BN = 1000
    xp, ft, fs = pl.pallas_call(
        _proj_kernel,
        grid=(N // BN,),
        in_specs=[pl.BlockSpec((BN, D), lambda i: (i, 0)),
                  pl.BlockSpec((D, HU), lambda i: (0, 0)),
                  pl.BlockSpec((HU, L), lambda i: (0, 0)),
                  pl.BlockSpec((HU, L), lambda i: (0, 0))],
        out_specs=[pl.BlockSpec((BN, HU), lambda i: (i, 0)),
                   pl.BlockSpec((BN, L), lambda i: (i, 0)),
                   pl.BlockSpec((BN, L), lambda i: (i, 0))],
        out_shape=[jax.ShapeDtypeStruct((N, HU), jnp.float32),
                   jax.ShapeDtypeStruct((N, L), jnp.float32),
                   jax.ShapeDtypeStruct((N, L), jnp.float32)],
    )(x, kernel, a1p, a2p)

    perm = np.empty((HU,), np.int32)
    for j in range(HU // 32):
        for u in range(16):
            perm[32 * j + 2 * u] = 32 * j + u
            perm[32 * j + 2 * u + 1] = 32 * j + 16 + u
    cs = jnp.take(xp, jnp.asarray(perm), axis=1).astype(jnp.bfloat16)

    tgt = edges[:, 1]
    src = edges[:, 0]

    ept = E // NS
    chmax = ept // L
    mesh = plsc.VectorSubcoreMesh(core_axis_name="c", subcore_axis_name="s")
    sc = pl.kernel(
        functools.partial(_sc_body, N, E),
        out_type=jax.ShapeDtypeStruct((N, CW), jnp.float32),
        mesh=mesh,
        compiler_params=pltpu.CompilerParams(
            use_tc_tiling_on_sc=False, needs_layout_passes=False),
        scratch_types=[
            pltpu.VMEM_SHARED((RN, CW), jnp.float32),
            pltpu.VMEM((ept,), jnp.int32),
            pltpu.VMEM((ept,), jnp.int32),
            pltpu.VMEM((chmax + 8, L), jnp.int32),
            pltpu.VMEM((chmax + 8, L), jnp.int32),
            pltpu.VMEM((NSLOT, L), jnp.int32),
            pltpu.VMEM((NSLOT, L, L), jnp.float32),
            pltpu.VMEM((NSLOT, L, L), jnp.float32),
            pltpu.VMEM((NSLOT, L, HU), jnp.bfloat16),
            pltpu.VMEM((2, L, CW), jnp.float32),
            pltpu.VMEM((L, CW), jnp.float32),
            pltpu.VMEM((HU,), jnp.float32),
            [pltpu.SemaphoreType.DMA] * NSLOT,
            [pltpu.SemaphoreType.DMA] * NSLOT,
            [pltpu.SemaphoreType.DMA] * NSLOT,
            [pltpu.SemaphoreType.DMA] * 2,
        ],
    )
    out = sc(tgt, src, ft, fs, cs, bias)
    return out[:, :HU]


# speedup vs baseline: 1.3319x; 1.3319x over previous
import functools

import jax
import jax.numpy as jnp
from jax import lax
from jax.experimental import pallas as pl
from jax.experimental.pallas import tpu as pltpu
from jax.experimental.pallas import tpu_sc as plsc

H = 8
U = 64
HU = H * U
CW = HU + 16
L = 16
NC = 2
NS = 16
RN = 1280
NRANGE = 8
NSLOT = 3


def _vgather(v, idx):
    dnums = lax.GatherDimensionNumbers(
        offset_dims=(), collapsed_slice_dims=(0,), start_index_map=(0,))
    return lax.gather(v, idx[:, None], dnums, (1,),
                      mode=lax.GatherScatterMode.PROMISE_IN_BOUNDS)


def _proj_kernel(x_ref, w_ref, a1_ref, a2_ref, cs_ref, ft_ref):
    xp = jnp.dot(x_ref[...], w_ref[...], preferred_element_type=jnp.float32)
    fs = jnp.dot(xp, a2_ref[...], preferred_element_type=jnp.float32)
    cs_ref[...] = jnp.concatenate([xp, fs], axis=1)
    ft_ref[...] = jnp.dot(xp, a1_ref[...], preferred_element_type=jnp.float32)


def _sc_body(N, E, tgt_hbm, src_hbm, ft_hbm, cs_hbm, bias_hbm,
             out_hbm, acc_sp, tgt_v, src_v, lsrc, ltgtl,
             tix, ftr, xpr, zrow, bias_v, gsem_f, gsem_c, ssem):
    ept = E // NS
    chmax = ept // L
    c = lax.axis_index("c")
    t = lax.axis_index("s")
    lanes = lax.iota(jnp.int32, L)
    zi = jnp.zeros((L,), jnp.int32)
    zf = jnp.zeros((L,), jnp.float32)
    hidx = [jnp.full((L,), h, jnp.int32) for h in range(H)]

    ebase = t * ept
    pltpu.sync_copy(tgt_hbm.at[pl.ds(ebase, ept)], tgt_v)
    pltpu.sync_copy(src_hbm.at[pl.ds(ebase, ept)], src_v)
    pltpu.sync_copy(bias_hbm, bias_v)

    def z_body(e, carry):
        for j in range(CW // L):
            zrow[e, pl.ds(j * L, L)] = zf
        return carry
    lax.fori_loop(0, L, z_body, jnp.int32(0))

    sbase = t * (RN // NS)

    def range_body(r, rcarry):
        lo = (c * (NRANGE // NC) + r) * RN

        def issue_gathers(k, s):
            tix[s, :] = ltgtl[k, :] + lo
            pltpu.async_copy(ft_hbm.at[tix.at[s]], ftr.at[s], gsem_f[s])
            pltpu.async_copy(cs_hbm.at[lsrc.at[k]], xpr.at[s], gsem_c[s])

        def zero_body(b, carry):
            pltpu.sync_copy(zrow, acc_sp.at[pl.ds(sbase + b * L, L)])
            return carry
        lax.fori_loop(0, RN // NS // L, zero_body, jnp.int32(0))
        plsc.subcore_barrier()

        @plsc.parallel_loop(0, chmax, 1, unroll=2, carry=jnp.int32(0))
        def scan_body(ci, cnt):
            off = pl.multiple_of(ci * L, L)
            tg = tgt_v[pl.ds(off, L)]
            sr = src_v[pl.ds(off, L)]
            tl = tg - lo
            m = (tl >= 0) & (tl < RN)
            pos = cnt + plsc.cumsum(m.astype(jnp.int32)) - 1
            row = lax.shift_right_logical(pos, 4)
            col = lax.bitwise_and(pos, 15)
            plsc.store_scatter(lsrc, [row, col], sr, mask=m)
            plsc.store_scatter(ltgtl, [row, col], tl, mask=m)
            return cnt + jnp.max(plsc.all_reduce_population_count(m))

        cnt = scan_body

        nch = lax.shift_right_logical(cnt + 15, 4)
        nchp = ((nch + NSLOT - 1) // NSLOT) * NSLOT

        def tz_body(b, carry):
            pos2 = cnt + b * L + lanes
            mtz = pos2 < (nchp + 1) * L
            plsc.store_scatter(
                lsrc, [lax.shift_right_logical(pos2, 4),
                       lax.bitwise_and(pos2, 15)], zi, mask=mtz)
            plsc.store_scatter(
                ltgtl, [lax.shift_right_logical(pos2, 4),
                        lax.bitwise_and(pos2, 15)], zi, mask=mtz)
            return carry
        lax.fori_loop(0, NSLOT + 2, tz_body, jnp.int32(0))

        issue_gathers(jnp.int32(0), 0)

        def trip_body(m, carry):
            for b in range(NSLOT):
                k = m * NSLOT + b
                s = b
                sn = (b + 1) % NSLOT

                @pl.when(k >= 2)
                def _():
                    pltpu.make_async_copy(xpr.at[sn], acc_sp.at[ltgtl.at[k]],
                                          ssem[sn]).wait()
                issue_gathers(k + 1, sn)
                pltpu.make_async_copy(ft_hbm.at[tix.at[s]], ftr.at[s],
                                      gsem_f[s]).wait()
                pltpu.make_async_copy(cs_hbm.at[lsrc.at[k]], xpr.at[s],
                                      gsem_c[s]).wait()

                base = k * L

                @plsc.parallel_loop(0, L, 1, unroll=4)
                def p_body(e):
                    rowv = ftr[s, e, :] + xpr[s, e, pl.ds(HU, L)]
                    sv = jnp.where(rowv > 0, rowv, 0.2 * rowv)
                    p = jnp.exp(sv)
                    p = jnp.where(base + e < cnt, p, 0.0)
                    xpr[s, e, pl.ds(HU, L)] = p
                    for h in range(H):
                        w = _vgather(p, hidx[h])
                        for q in range(HU // L // H):
                            j = h * (HU // L // H) + q
                            xpr[s, e, pl.ds(j * L, L)] = (
                                w * xpr[s, e, pl.ds(j * L, L)])
                pltpu.async_copy(xpr.at[s], acc_sp.at[ltgtl.at[k]],
                                 ssem[s], add=True)
            return carry

        nts = nchp // NSLOT
        lax.fori_loop(0, nts, trip_body, jnp.int32(0))

        pltpu.make_async_copy(ft_hbm.at[tix.at[0]], ftr.at[0],
                              gsem_f[0]).wait()
        pltpu.make_async_copy(cs_hbm.at[lsrc.at[0]], xpr.at[0],
                              gsem_c[0]).wait()

        @pl.when(nchp > 0)
        def _():
            pltpu.make_async_copy(xpr.at[1], acc_sp.at[ltgtl.at[0]],
                                  ssem[1]).wait()
            pltpu.make_async_copy(xpr.at[2], acc_sp.at[ltgtl.at[0]],
                                  ssem[2]).wait()
        plsc.subcore_barrier()

        def ep_body(b, carry):
            bstart = sbase + b * L
            gs = lo + bstart

            @pl.when(gs < N)
            def _():
                pltpu.sync_copy(acc_sp.at[pl.ds(bstart, L)], xpr.at[0])

                @plsc.parallel_loop(0, L, 1, unroll=2)
                def row_body(e):
                    srow = xpr[0, e, pl.ds(HU, L)]
                    for h in range(H):
                        sv = _vgather(srow, hidx[h]) + 1e-7
                        inv = 1.0 / sv
                        for q in range(HU // L // H):
                            j = h * (HU // L // H) + q
                            z = (xpr[0, e, pl.ds(j * L, L)] * inv
                                 + bias_v[pl.ds(j * L, L)])
                            xpr[0, e, pl.ds(j * L, L)] = jnp.where(
                                z > 0, z, jnp.exp(z) - 1.0)
                pltpu.sync_copy(xpr.at[0], out_hbm.at[pl.ds(gs, L)])
            return carry
        lax.fori_loop(0, RN // NS // L, ep_body, jnp.int32(0))
        plsc.subcore_barrier()
        return rcarry
    lax.fori_loop(0, NRANGE // NC, range_body, jnp.int32(0))


def kernel(x, edges, kernel, kernel_attention1, kernel_attention2, bias,
           training):
    N, D = x.shape
    E = edges.shape[0]

    eye = jnp.eye(H, dtype=x.dtype)
    a1 = (kernel_attention1[0][:, :, None] * eye[:, None, :]).reshape(HU, H)
    a2 = (kernel_attention2[0][:, :, None] * eye[:, None, :]).reshape(HU, H)
    pad = jnp.zeros((HU, L - H), jnp.float32)
    a1p = jnp.concatenate([a1, pad], axis=1)
    a2p = jnp.concatenate([a2, pad], axis=1)

    BN = 1000
    cs, ft = pl.pallas_call(
        _proj_kernel,
        grid=(N // BN,),
        in_specs=[pl.BlockSpec((BN, D), lambda i: (i, 0)),
                  pl.BlockSpec((D, HU), lambda i: (0, 0)),
                  pl.BlockSpec((HU, L), lambda i: (0, 0)),
                  pl.BlockSpec((HU, L), lambda i: (0, 0))],
        out_specs=[pl.BlockSpec((BN, CW), lambda i: (i, 0)),
                   pl.BlockSpec((BN, L), lambda i: (i, 0))],
        out_shape=[jax.ShapeDtypeStruct((N, CW), jnp.float32),
                   jax.ShapeDtypeStruct((N, L), jnp.float32)],
    )(x, kernel, a1p, a2p)

    tgt = edges[:, 1]
    src = edges[:, 0]

    ept = E // NS
    chmax = ept // L
    mesh = plsc.VectorSubcoreMesh(core_axis_name="c", subcore_axis_name="s")
    sc = pl.kernel(
        functools.partial(_sc_body, N, E),
        out_type=jax.ShapeDtypeStruct((N, CW), jnp.float32),
        mesh=mesh,
        compiler_params=pltpu.CompilerParams(
            use_tc_tiling_on_sc=False, needs_layout_passes=False),
        scratch_types=[
            pltpu.VMEM_SHARED((RN, CW), jnp.float32),
            pltpu.VMEM((ept,), jnp.int32),
            pltpu.VMEM((ept,), jnp.int32),
            pltpu.VMEM((chmax + 8, L), jnp.int32),
            pltpu.VMEM((chmax + 8, L), jnp.int32),
            pltpu.VMEM((NSLOT, L), jnp.int32),
            pltpu.VMEM((NSLOT, L, L), jnp.float32),
            pltpu.VMEM((NSLOT, L, CW), jnp.float32),
            pltpu.VMEM((L, CW), jnp.float32),
            pltpu.VMEM((HU,), jnp.float32),
            [pltpu.SemaphoreType.DMA] * NSLOT,
            [pltpu.SemaphoreType.DMA] * NSLOT,
            [pltpu.SemaphoreType.DMA] * NSLOT,
        ],
    )
    out = sc(tgt, src, ft, cs, bias)
    return out[:, :HU]
